# Initial kernel scaffold; baseline (speedup 1.0000x reference)
import jax
import jax.numpy as jnp
from jax.experimental import pallas as pl


def kernel(predicted_patches, target, mask):
    raise NotImplementedError("write your pallas kernel here")


# speedup vs baseline: 2.8719x; 2.8719x over previous
import jax
import jax.numpy as jnp
import numpy as np
from jax.experimental import pallas as pl

_B = 64
_C = 3
_BI = 3
_H = 512
_PS = 16
_NP = _H // _PS

_rows = np.arange(_C * _NP)
_cols = np.arange(_C * _H)
_LMAT = (
    ((_cols[None, :] // _H) == (_rows[:, None] // _NP))
    & (((_cols[None, :] % _H) // _PS) == (_rows[:, None] % _NP))
).astype(np.float32)
_RMAT = ((np.arange(_H)[:, None] // _PS) == np.arange(_NP)[None, :]).astype(
    np.float32
)


def _mpp_kernel(tgt_ref, pred_ref, mask_ref, lmat_ref, rmat_ref, out_ref):
    b = pl.program_id(0)

    t = tgt_ref[0]
    tp = jnp.dot(t, rmat_ref[...], preferred_element_type=jnp.float32)
    s = jnp.dot(lmat_ref[...], tp, preferred_element_type=jnp.float32)
    avg = s * (1.0 / (_PS * _PS))

    idx = (
        (avg >= 0.333).astype(jnp.int32)
        + (avg >= 0.666).astype(jnp.int32)
        + (avg >= 1.0).astype(jnp.int32)
    )

    m = mask_ref[0]
    m3 = jnp.broadcast_to(m[None], (_C, _NP, _NP)).reshape(_C * _NP, _NP)

    loss = jnp.zeros((1, 1), jnp.float32)
    for k in range(_BI):
        oh = (idx == k).astype(jnp.float32)
        d = pred_ref[0, k] - oh
        loss = loss + jnp.sum(d * d * m3, keepdims=True)
    cnt = jnp.sum(m, keepdims=True)

    @pl.when(b == 0)
    def _():
        out_ref[...] = jnp.zeros_like(out_ref)

    out_ref[0:1, 0:1] += loss
    out_ref[0:1, 1:2] += cnt

    @pl.when(b == pl.num_programs(0) - 1)
    def _():
        den = jnp.maximum(out_ref[0:1, 1:2] * (_C * _BI), 1.0)
        out_ref[0:1, 0:1] = out_ref[0:1, 0:1] / den


def kernel(predicted_patches, target, mask):
    tgt = target.reshape(_B, _C * _H, _H)
    pred = predicted_patches.reshape(_B, _NP, _NP, _C, _BI)
    pred = jnp.transpose(pred, (0, 4, 3, 1, 2)).reshape(_B, _BI, _C * _NP, _NP)
    mask_f = mask.astype(jnp.float32).reshape(_B, _NP, _NP)

    out = pl.pallas_call(
        _mpp_kernel,
        grid=(_B,),
        in_specs=[
            pl.BlockSpec((1, _C * _H, _H), lambda b: (b, 0, 0)),
            pl.BlockSpec((1, _BI, _C * _NP, _NP), lambda b: (b, 0, 0, 0)),
            pl.BlockSpec((1, _NP, _NP), lambda b: (b, 0, 0)),
            pl.BlockSpec((_C * _NP, _C * _H), lambda b: (0, 0)),
            pl.BlockSpec((_H, _NP), lambda b: (0, 0)),
        ],
        out_specs=pl.BlockSpec((1, 2), lambda b: (0, 0)),
        out_shape=jax.ShapeDtypeStruct((1, 2), jnp.float32),
    )(tgt, pred, mask_f, jnp.asarray(_LMAT), jnp.asarray(_RMAT))
    return out[0, 0]
